# Initial kernel scaffold; baseline (speedup 1.0000x reference)
import jax
import jax.numpy as jnp
from jax.experimental import pallas as pl


def kernel(node_batch, batch_emb, all_emb, sample_pos, sample_neg):
    raise NotImplementedError("write your pallas kernel here")


# speedup vs baseline: 6.5131x; 6.5131x over previous
import functools

import jax
import jax.numpy as jnp
import numpy as np
from jax import lax
from jax.experimental import pallas as pl
from jax.experimental.pallas import tpu as pltpu
import jax.experimental.pallas.tpu_sc as plsc

_S = 10
_NC = 2
_NS = 16
_NW = _NC * _NS
_L = 16
_CH = 128


def _dyn_gather(x, idx):
  return lax.gather(
      x, idx[:, None],
      lax.GatherDimensionNumbers(
          offset_dims=(), collapsed_slice_dims=(0,), start_index_map=(0,)),
      (1,), mode=lax.GatherScatterMode.PROMISE_IN_BOUNDS)


def _sc_gather(nb, rand_flat_pos, rand_flat_neg, sp_flat, sn_flat, all_emb,
               max_pos, max_neg):
  B = nb.shape[0]
  N, D = all_emb.shape
  RW = B // _NW
  SR = RW * _S
  NCH = SR // _CH

  mesh = plsc.VectorSubcoreMesh(
      core_axis_name="c", subcore_axis_name="s",
      num_cores=_NC, num_subcores=_NS)

  @functools.partial(
      pl.kernel,
      out_type=(jax.ShapeDtypeStruct((B * _S, D), jnp.float32),
                jax.ShapeDtypeStruct((B * _S, D), jnp.float32)),
      mesh=mesh,
      scratch_types=[
          pltpu.VMEM((B,), jnp.int32),
          pltpu.VMEM((N,), jnp.int32),
          pltpu.VMEM((RW,), jnp.int32),
          pltpu.VMEM((SR,), jnp.int32),
          pltpu.VMEM((SR,), jnp.int32),
          pltpu.VMEM((SR,), jnp.int32),
          pltpu.VMEM((SR,), jnp.int32),
          pltpu.VMEM((SR,), jnp.int32),
          pltpu.VMEM((SR,), jnp.int32),
          pltpu.VMEM((_CH, D), jnp.float32),
          pltpu.VMEM((_CH, D), jnp.float32),
          pltpu.SemaphoreType.DMA,
          pltpu.SemaphoreType.DMA,
          pltpu.SemaphoreType.DMA,
      ],
      compiler_params=pltpu.CompilerParams(needs_layout_passes=False),
  )
  def k(nb_hbm, rp_hbm, rn_hbm, spf_hbm, snf_hbm, emb_hbm,
        pos_out, neg_out,
        nbv, win, wv, ridx, nbase, rvp, rvn, pidx, nidx,
        ebuf0, ebuf1, sem, semA, semB):
    wid = lax.axis_index("s") * _NC + lax.axis_index("c")
    base = wid * RW

    pltpu.sync_copy(nb_hbm, nbv)

    lanes = lax.iota(jnp.int32, _L)

    def win_body(p, carry):
      v = nbv[pl.ds(p * _L, _L)]
      ml = lanes
      for kk in range(1, _L):
        rot = lax.rem(lanes + kk, _L)
        vr = _dyn_gather(v, rot)
        ml = jnp.where(vr == v, jnp.maximum(ml, rot), ml)
      plsc.store_scatter(win, [v], p * _L + ml)
      return carry
    lax.fori_loop(0, B // _L, win_body, 0)

    for q in range(RW // _L):
      idxv = nbv[pl.ds(base + q * _L, _L)]
      wv[pl.ds(q * _L, _L)] = plsc.load_gather(win, [idxv])

    for v in range(SR // _L):
      fl = lanes + v * _L
      il = lax.div(fl, _S)
      sl = lax.rem(fl, _S)
      wl = plsc.load_gather(wv, [il])
      ridx[pl.ds(v * _L, _L)] = wl * _S + sl
      nbase[pl.ds(v * _L, _L)] = plsc.load_gather(nbv, [base + il])

    cps = []
    for c in range(NCH):
      s_ = pl.ds(c * _CH, _CH)
      cps.append(pltpu.async_copy(rp_hbm.at[ridx.at[s_]], rvp.at[s_], sem))
      cps.append(pltpu.async_copy(rn_hbm.at[ridx.at[s_]], rvn.at[s_], sem))
    for cp in cps:
      cp.wait()

    for v in range(SR // _L):
      s_ = pl.ds(v * _L, _L)
      nb_l = nbase[s_]
      ridx[s_] = nb_l * max_pos + rvp[s_]
      nbase[s_] = nb_l * max_neg + rvn[s_]
    cps = []
    for c in range(NCH):
      s_ = pl.ds(c * _CH, _CH)
      cps.append(pltpu.async_copy(spf_hbm.at[ridx.at[s_]], pidx.at[s_], sem))
      cps.append(pltpu.async_copy(snf_hbm.at[nbase.at[s_]], nidx.at[s_], sem))
    for cp in cps:
      cp.wait()

    bufs = (ebuf0, ebuf1)
    sems = (semA, semB)
    prev = None
    for c in range(2 * NCH):
      idx_ref = (pidx if c < NCH else nidx).at[pl.ds((c % NCH) * _CH, _CH)]
      out_ref = pos_out if c < NCH else neg_out
      row0 = base * _S + (c % NCH) * _CH
      buf = bufs[c % 2]
      d = pltpu.async_copy(emb_hbm.at[idx_ref], buf, sems[c % 2])
      if prev is not None:
        pd, pbuf, pout, prow = prev
        pd.wait()
        pltpu.sync_copy(pbuf, pout.at[pl.ds(prow, _CH)])
      prev = (d, buf, out_ref, row0)
    pd, pbuf, pout, prow = prev
    pd.wait()
    pltpu.sync_copy(pbuf, pout.at[pl.ds(prow, _CH)])

  return k(nb, rand_flat_pos, rand_flat_neg, sp_flat, sn_flat, all_emb)


def _softplus(z):
  return jnp.maximum(z, 0.0) + jnp.log1p(jnp.exp(-jnp.abs(z)))


def _tc_loss(batch_emb, pos_flat, neg_flat):
  B, D = batch_emb.shape
  R = 256
  G = B // R

  def body(be_ref, pg_ref, ng_ref, out_ref):
    i = pl.program_id(0)
    be = be_ref[...]
    step = jnp.float32(0.0)
    for s in range(_S):
      dp = jnp.sum(be * pg_ref[:, s * D:(s + 1) * D], axis=1)
      dn = jnp.sum(be * ng_ref[:, s * D:(s + 1) * D], axis=1)
      step = step + jnp.sum(_softplus(-dp) + _softplus(dn))

    @pl.when(i == 0)
    def _():
      out_ref[0, 0] = 0.0
    out_ref[0, 0] += step

  out = pl.pallas_call(
      body,
      grid=(G,),
      in_specs=[
          pl.BlockSpec((R, D), lambda i: (i, 0)),
          pl.BlockSpec((R, _S * D), lambda i: (i, 0)),
          pl.BlockSpec((R, _S * D), lambda i: (i, 0)),
      ],
      out_specs=pl.BlockSpec((1, 1), lambda i: (0, 0),
                             memory_space=pltpu.SMEM),
      out_shape=jax.ShapeDtypeStruct((1, 1), jnp.float32),
  )(batch_emb, pos_flat, neg_flat)
  return out.reshape(1) / B


def kernel(node_batch, batch_emb, all_emb, sample_pos, sample_neg):
  B = node_batch.shape[0]
  N, D = all_emb.shape
  max_pos = sample_pos.shape[1]
  max_neg = sample_neg.shape[1]

  rk = jax.random.key(42)
  r1, r2 = jax.random.split(rk)
  rand_pos = jax.random.randint(r1, (B, _S), 0, max_pos)
  rand_neg = jax.random.randint(r2, (B, _S), 0, max_neg)

  nb = node_batch.astype(jnp.int32)
  rp = rand_pos.astype(jnp.int32).reshape(-1)
  rn = rand_neg.astype(jnp.int32).reshape(-1)
  spf = sample_pos.astype(jnp.int32).reshape(-1)
  snf = sample_neg.astype(jnp.int32).reshape(-1)
  ae = all_emb.astype(jnp.float32)
  be = batch_emb.astype(jnp.float32)

  pos_g, neg_g = _sc_gather(nb, rp, rn, spf, snf, ae, max_pos, max_neg)
  return _tc_loss(be, pos_g.reshape(B, _S * D), neg_g.reshape(B, _S * D))


# speedup vs baseline: 10.5226x; 1.6156x over previous
import functools

import jax
import jax.numpy as jnp
import numpy as np
from jax import lax
from jax.experimental import pallas as pl
from jax.experimental.pallas import tpu as pltpu
import jax.experimental.pallas.tpu_sc as plsc

_S = 10
_NC = 2
_NS = 16
_NW = _NC * _NS
_L = 16
_CH = 128


def _dyn_gather(x, idx):
  return lax.gather(
      x, idx[:, None],
      lax.GatherDimensionNumbers(
          offset_dims=(), collapsed_slice_dims=(0,), start_index_map=(0,)),
      (1,), mode=lax.GatherScatterMode.PROMISE_IN_BOUNDS)


def _sc_dots(nb, rand_flat_pos, rand_flat_neg, sp_flat, sn_flat, all_emb,
             batch_emb, max_pos, max_neg):
  B = nb.shape[0]
  N, D = all_emb.shape
  RW = B // _NW
  SR = RW * _S
  NCH = SR // _CH
  DK = D // _L

  mesh = plsc.VectorSubcoreMesh(
      core_axis_name="c", subcore_axis_name="s",
      num_cores=_NC, num_subcores=_NS)

  @functools.partial(
      pl.kernel,
      out_type=(jax.ShapeDtypeStruct((B * _S,), jnp.float32),
                jax.ShapeDtypeStruct((B * _S,), jnp.float32)),
      mesh=mesh,
      scratch_types=[
          pltpu.VMEM((B,), jnp.int32),
          pltpu.VMEM((N,), jnp.int32),
          pltpu.VMEM((RW,), jnp.int32),
          pltpu.VMEM((SR,), jnp.int32),
          pltpu.VMEM((SR,), jnp.int32),
          pltpu.VMEM((SR,), jnp.int32),
          pltpu.VMEM((SR,), jnp.int32),
          pltpu.VMEM((2 * SR,), jnp.int32),
          pltpu.VMEM((RW, D), jnp.float32),
          pltpu.VMEM((2, _CH, D), jnp.float32),
          pltpu.VMEM((2 * SR,), jnp.float32),
          pltpu.VMEM((_L * _L,), jnp.float32),
          pltpu.SemaphoreType.DMA,
          pltpu.SemaphoreType.DMA,
          pltpu.SemaphoreType.DMA,
          pltpu.SemaphoreType.DMA,
      ],
      compiler_params=pltpu.CompilerParams(needs_layout_passes=False),
  )
  def k(nb_hbm, rp_hbm, rn_hbm, spf_hbm, snf_hbm, emb_hbm, be_hbm,
        pos_out, neg_out,
        nbv, win, wv, ridx, nbase, rvp, rvn, sidx, bel,
        ebuf, dots, ptmp, sem, semA, semB, semC):
    wid = lax.axis_index("s") * _NC + lax.axis_index("c")
    base = wid * RW

    be_cp = pltpu.async_copy(be_hbm.at[pl.ds(base, RW)], bel, semC)
    pltpu.sync_copy(nb_hbm, nbv)

    lanes = lax.iota(jnp.int32, _L)

    def win_body(p, carry):
      v = nbv[pl.ds(p * _L, _L)]
      ml = lanes
      for kk in range(1, _L):
        rot = lax.rem(lanes + kk, _L)
        vr = _dyn_gather(v, rot)
        ml = jnp.where(vr == v, jnp.maximum(ml, rot), ml)
      plsc.store_scatter(win, [v], p * _L + ml)
      return carry
    lax.fori_loop(0, B // _L, win_body, 0)

    for q in range(RW // _L):
      idxv = nbv[pl.ds(base + q * _L, _L)]
      wv[pl.ds(q * _L, _L)] = plsc.load_gather(win, [idxv])

    for v in range(SR // _L):
      fl = lanes + v * _L
      il = lax.div(fl, _S)
      sl = lax.rem(fl, _S)
      wl = plsc.load_gather(wv, [il])
      ridx[pl.ds(v * _L, _L)] = wl * _S + sl
      nbase[pl.ds(v * _L, _L)] = plsc.load_gather(nbv, [base + il])

    cps = []
    for c in range(NCH):
      s_ = pl.ds(c * _CH, _CH)
      cps.append(pltpu.async_copy(rp_hbm.at[ridx.at[s_]], rvp.at[s_], sem))
      cps.append(pltpu.async_copy(rn_hbm.at[ridx.at[s_]], rvn.at[s_], sem))
    for cp in cps:
      cp.wait()

    for v in range(SR // _L):
      s_ = pl.ds(v * _L, _L)
      nb_l = nbase[s_]
      ridx[s_] = nb_l * max_pos + rvp[s_]
      nbase[s_] = nb_l * max_neg + rvn[s_]
    cps = []
    for c in range(NCH):
      s_ = pl.ds(c * _CH, _CH)
      cps.append(pltpu.async_copy(
          spf_hbm.at[ridx.at[s_]], sidx.at[s_], sem))
      cps.append(pltpu.async_copy(
          snf_hbm.at[nbase.at[s_]], sidx.at[pl.ds(SR + c * _CH, _CH)], sem))
    for cp in cps:
      cp.wait()
    be_cp.wait()

    NCH2 = 2 * NCH
    lanes16 = lanes * _L

    def process(c, b, semX):
      pltpu.make_async_copy(
          emb_hbm.at[sidx.at[pl.ds(c * _CH, _CH)]], ebuf.at[b], semX).wait()

      def group_body(g, carry):
        r0 = g * _L
        fr0 = c * _CH + r0
        for rr in range(_L):
          fr = fr0 + rr
          il = lax.div(lax.rem(fr, SR), _S)
          acc = ebuf[b, r0 + rr, pl.ds(0, _L)] * bel[il, pl.ds(0, _L)]
          for kk in range(1, DK):
            acc = acc + (ebuf[b, r0 + rr, pl.ds(kk * _L, _L)] *
                         bel[il, pl.ds(kk * _L, _L)])
          ptmp[pl.ds(rr * _L, _L)] = acc
        tot = plsc.load_gather(ptmp, [lanes16])
        for kk in range(1, _L):
          tot = tot + plsc.load_gather(ptmp, [lanes16 + kk])
        dots[pl.ds(fr0, _L)] = tot
        return carry
      lax.fori_loop(0, _CH // _L, group_body, 0)

      @pl.when(c < NCH2 - 2)
      def _():
        pltpu.async_copy(
            emb_hbm.at[sidx.at[pl.ds((c + 2) * _CH, _CH)]], ebuf.at[b], semX)

    pltpu.async_copy(emb_hbm.at[sidx.at[pl.ds(0, _CH)]], ebuf.at[0], semA)
    pltpu.async_copy(emb_hbm.at[sidx.at[pl.ds(_CH, _CH)]], ebuf.at[1], semB)

    def chunk_body(c, carry):
      @pl.when(lax.rem(c, 2) == 0)
      def _():
        process(c, 0, semA)

      @pl.when(lax.rem(c, 2) == 1)
      def _():
        process(c, 1, semB)
      return carry
    lax.fori_loop(0, NCH2, chunk_body, 0)

    pltpu.sync_copy(dots.at[pl.ds(0, SR)], pos_out.at[pl.ds(base * _S, SR)])
    pltpu.sync_copy(dots.at[pl.ds(SR, SR)], neg_out.at[pl.ds(base * _S, SR)])

  return k(nb, rand_flat_pos, rand_flat_neg, sp_flat, sn_flat, all_emb,
           batch_emb)


def _softplus(z):
  return jnp.maximum(z, 0.0) + jnp.log1p(jnp.exp(-jnp.abs(z)))


def _tc_loss(dots_p, dots_n):
  n = dots_p.shape[0]
  R = n // 128

  def body(dp_ref, dn_ref, out_ref):
    tot = jnp.sum(_softplus(-dp_ref[...]) + _softplus(dn_ref[...]))
    out_ref[0, 0] = tot

  out = pl.pallas_call(
      body,
      out_specs=pl.BlockSpec(memory_space=pltpu.SMEM),
      out_shape=jax.ShapeDtypeStruct((1, 1), jnp.float32),
  )(dots_p.reshape(R, 128), dots_n.reshape(R, 128))
  return out.reshape(1)


def kernel(node_batch, batch_emb, all_emb, sample_pos, sample_neg):
  B = node_batch.shape[0]
  N, D = all_emb.shape
  max_pos = sample_pos.shape[1]
  max_neg = sample_neg.shape[1]

  rk = jax.random.key(42)
  r1, r2 = jax.random.split(rk)
  rand_pos = jax.random.randint(r1, (B, _S), 0, max_pos)
  rand_neg = jax.random.randint(r2, (B, _S), 0, max_neg)

  nb = node_batch.astype(jnp.int32)
  rp = rand_pos.astype(jnp.int32).reshape(-1)
  rn = rand_neg.astype(jnp.int32).reshape(-1)
  spf = sample_pos.astype(jnp.int32).reshape(-1)
  snf = sample_neg.astype(jnp.int32).reshape(-1)
  ae = all_emb.astype(jnp.float32)
  be = batch_emb.astype(jnp.float32)

  dp, dn = _sc_dots(nb, rp, rn, spf, snf, ae, be, max_pos, max_neg)
  return _tc_loss(dp, dn) / B
